# baseline (speedup 1.0000x reference)
import functools

import jax
import jax.numpy as jnp
from jax import lax
from jax.experimental import pallas as pl
from jax.experimental.pallas import tpu as pltpu
from jax.experimental.pallas import tpu_sc as plsc

EMBEDDING_DIM = 32
BATCH = 16384
FIELDS = 26
B_TOTAL = BATCH * FIELDS
NUM_EMB = 1000000

NUM_WORKERS = 32

TCOLS = 512
NFULL = NUM_EMB // TCOLS
NPIPE = 61
EXTRA_CHUNK = NPIPE * NUM_WORKERS
FULL_COLS = NFULL * TCOLS
TAIL = NUM_EMB - FULL_COLS

_mesh = plsc.VectorSubcoreMesh(core_axis_name="c", subcore_axis_name="s")


@functools.partial(
    pl.kernel,
    mesh=_mesh,
    out_type=jax.ShapeDtypeStruct((NUM_EMB * EMBEDDING_DIM,), jnp.float32),
    scratch_types=[
        pltpu.VMEM((EMBEDDING_DIM, TCOLS), jnp.float32),
        pltpu.VMEM((EMBEDDING_DIM, TCOLS), jnp.float32),
        pltpu.VMEM((TCOLS * EMBEDDING_DIM,), jnp.float32),
        pltpu.VMEM((TCOLS * EMBEDDING_DIM,), jnp.float32),
        pltpu.VMEM((TCOLS * (EMBEDDING_DIM + 1),), jnp.float32),
        pltpu.SemaphoreType.DMA,
        pltpu.SemaphoreType.DMA,
        pltpu.SemaphoreType.DMA,
        pltpu.SemaphoreType.DMA,
    ],
    compiler_params=pltpu.CompilerParams(
        use_tc_tiling_on_sc=True, needs_layout_passes=False
    ),
)
def _transpose_kernel(
    wt_hbm, tail_hbm, wlin_hbm, slab0, slab1, tbuf0, tbuf1, skew, si0, si1, so0, so1
):
    slab = (slab0, slab1)
    tbuf = (tbuf0, tbuf1)
    si = (si0, si1)
    so = (so0, so1)
    wid = lax.axis_index("s") * 2 + lax.axis_index("c")

    lane = jax.lax.broadcasted_iota(jnp.int32, (16,), 0)
    lane32 = lane * EMBEDDING_DIM
    SKEW = EMBEDDING_DIM + 1
    lane33 = lane * SKEW

    def fire_load(i, b):
        c0 = (wid + NUM_WORKERS * i) * TCOLS
        return pltpu.async_copy(
            wt_hbm.at[:, pl.ds(c0, TCOLS)], slab[b], si[b]
        )

    rot = [(lane + d) & 15 for d in range(16)]
    cvec = [16 * cg + lane for cg in range(2)]

    def transpose_chunk(b, ncols):
        @plsc.parallel_loop(0, ncols // 16, unroll=2)
        def _(r0g):
            r0 = r0g * 16
            for cg in range(2):
                for d in range(16):
                    v = plsc.load_gather(slab[b], [cvec[cg], r0 + rot[d]])
                    plsc.store_scatter(
                        tbuf[b],
                        [(r0 + rot[d]) * EMBEDDING_DIM + cvec[cg]],
                        v,
                    )

    def fire_store(i, b):
        off = (wid + NUM_WORKERS * i) * (TCOLS * EMBEDDING_DIM)
        return pltpu.async_copy(
            tbuf[b], wlin_hbm.at[pl.ds(off, TCOLS * EMBEDDING_DIM)], so[b]
        )

    def wait_load(b):
        pltpu.make_async_copy(wt_hbm.at[:, pl.ds(0, TCOLS)], slab[b], si[b]).wait()

    def wait_store(b):
        pltpu.make_async_copy(
            tbuf[b], wlin_hbm.at[pl.ds(0, TCOLS * EMBEDDING_DIM)], so[b]
        ).wait()

    fire_load(0, 0)
    fire_load(1, 1)

    def chunk_body(c, carry):
        def process(b):
            @pl.when(c >= 2)
            def _():
                wait_store(b)

            wait_load(b)
            transpose_chunk(b, TCOLS)

            @pl.when(c + 2 < NPIPE)
            def _():
                fire_load(c + 2, b)

            fire_store(c, b)

        @pl.when(lax.rem(c, 2) == 0)
        def _():
            process(0)

        @pl.when(lax.rem(c, 2) == 1)
        def _():
            process(1)

        return carry

    lax.fori_loop(0, NPIPE, chunk_body, 0)
    for b in range(2):
        wait_store(b)

    @pl.when(wid == 0)
    def _():
        pltpu.sync_copy(
            wt_hbm.at[:, pl.ds(EXTRA_CHUNK * TCOLS, TCOLS)], slab[0]
        )
        transpose_chunk(0, TCOLS)
        pltpu.sync_copy(
            tbuf[0],
            wlin_hbm.at[
                pl.ds(EXTRA_CHUNK * TCOLS * EMBEDDING_DIM, TCOLS * EMBEDDING_DIM)
            ],
        )

    @pl.when(wid == 1)
    def _():
        pltpu.sync_copy(
            tail_hbm, tbuf[0].at[pl.ds(0, TAIL * EMBEDDING_DIM)]
        )
        pltpu.sync_copy(
            tbuf[0].at[pl.ds(0, TAIL * EMBEDDING_DIM)],
            wlin_hbm.at[pl.ds(FULL_COLS * EMBEDDING_DIM, TAIL * EMBEDDING_DIM)],
        )


B_PER_W = BATCH // NUM_WORKERS
GB = 64
GCHUNK = GB * FIELDS
NGC = B_PER_W // GB
STG_MINOR = GB + 1


@functools.partial(
    pl.kernel,
    mesh=_mesh,
    out_type=jax.ShapeDtypeStruct(
        (FIELDS, 4, BATCH // 128, 8, 128), jnp.float32
    ),
    scratch_types=[
        pltpu.VMEM((GCHUNK,), jnp.int32),
        pltpu.VMEM((GCHUNK, EMBEDDING_DIM), jnp.float32),
        pltpu.VMEM((FIELDS, 4, 8, STG_MINOR), jnp.float32),
        pltpu.SemaphoreType.DMA,
        pltpu.SemaphoreType.DMA,
    ],
    compiler_params=pltpu.CompilerParams(
        use_tc_tiling_on_sc=False, needs_layout_passes=False
    ),
)
def _gather_kernel(idx_hbm, table_hbm, out_hbm, idx_v, rows_v, stg_v, sg, sw):
    wid = lax.axis_index("s") * 2 + lax.axis_index("c")
    lane = jax.lax.broadcasted_iota(jnp.int32, (16,), 0)
    ci_vec = lane & 7
    br_vec = (lane >> 3, 2 + (lane >> 3))

    def fire_gather(c):
        j0 = (wid * B_PER_W + c * GB) * FIELDS
        pltpu.sync_copy(idx_hbm.at[pl.ds(j0, GCHUNK)], idx_v)
        return pltpu.async_copy(table_hbm.at[idx_v], rows_v, sg)

    def wait_gather():
        pltpu.make_async_copy(table_hbm.at[idx_v], rows_v, sg).wait()

    fire_gather(0)

    def chunk_body(c, carry):
        wait_gather()

        @plsc.parallel_loop(0, GB, unroll=2)
        def _(bi):
            bi_vec = lane * 0 + bi
            for f in range(FIELDS):
                f_vec = lane * 0 + f
                for g in range(2):
                    v = rows_v[bi * FIELDS + f, pl.ds(16 * g, 16)]
                    plsc.store_scatter(
                        stg_v, [f_vec, br_vec[g], ci_vec, bi_vec], v
                    )

        bc = wid * (B_PER_W // 128) + c // 2
        bi0 = (c % 2) * GB
        copies = []
        for f in range(FIELDS):
            copies.append(
                pltpu.async_copy(
                    stg_v.at[f, :, :, pl.ds(0, GB)],
                    out_hbm.at[f, :, bc, :, pl.ds(bi0, GB)],
                    sw,
                )
            )

        @pl.when(c + 1 < NGC)
        def _():
            fire_gather(c + 1)

        for cp in copies:
            cp.wait()
        return carry

    lax.fori_loop(0, NGC, chunk_body, 0)


def kernel(x, weight):
    tail = weight[FULL_COLS:].reshape(-1)
    wlin = _transpose_kernel(weight.T, tail)
    table = wlin.reshape(NUM_EMB, EMBEDDING_DIM)
    idx = x.reshape(-1)
    o5 = _gather_kernel(idx, table)
    return o5.transpose(2, 4, 0, 1, 3).reshape(BATCH, FIELDS, EMBEDDING_DIM)


# speedup vs baseline: 1.9012x; 1.9012x over previous
import functools

import jax
import jax.numpy as jnp
from jax import lax
from jax.experimental import pallas as pl
from jax.experimental.pallas import tpu as pltpu
from jax.experimental.pallas import tpu_sc as plsc

EMBEDDING_DIM = 32
BATCH = 16384
FIELDS = 26
B_TOTAL = BATCH * FIELDS
NUM_EMB = 1000000

NUM_WORKERS = 32

TCOLS = 512
NFULL = NUM_EMB // TCOLS
NPIPE = 61
EXTRA_CHUNK = NPIPE * NUM_WORKERS
FULL_COLS = NFULL * TCOLS
TAIL = NUM_EMB - FULL_COLS

_mesh = plsc.VectorSubcoreMesh(core_axis_name="c", subcore_axis_name="s")


@functools.partial(
    pl.kernel,
    mesh=_mesh,
    out_type=jax.ShapeDtypeStruct((NUM_EMB * EMBEDDING_DIM,), jnp.float32),
    scratch_types=[
        pltpu.VMEM((EMBEDDING_DIM, TCOLS), jnp.float32),
        pltpu.VMEM((EMBEDDING_DIM, TCOLS), jnp.float32),
        pltpu.VMEM((TCOLS * EMBEDDING_DIM,), jnp.float32),
        pltpu.VMEM((TCOLS * EMBEDDING_DIM,), jnp.float32),
        pltpu.VMEM((TCOLS * (EMBEDDING_DIM + 1),), jnp.float32),
        pltpu.SemaphoreType.DMA,
        pltpu.SemaphoreType.DMA,
        pltpu.SemaphoreType.DMA,
        pltpu.SemaphoreType.DMA,
    ],
    compiler_params=pltpu.CompilerParams(
        use_tc_tiling_on_sc=True, needs_layout_passes=False
    ),
)
def _transpose_kernel(
    wt_hbm, tail_hbm, wlin_hbm, slab0, slab1, tbuf0, tbuf1, skew, si0, si1, so0, so1
):
    slab = (slab0, slab1)
    tbuf = (tbuf0, tbuf1)
    si = (si0, si1)
    so = (so0, so1)
    wid = lax.axis_index("s") * 2 + lax.axis_index("c")

    lane = jax.lax.broadcasted_iota(jnp.int32, (16,), 0)
    lane32 = lane * EMBEDDING_DIM
    SKEW = EMBEDDING_DIM + 1
    lane33 = lane * SKEW

    def fire_load(i, b):
        c0 = (wid + NUM_WORKERS * i) * TCOLS
        return pltpu.async_copy(
            wt_hbm.at[:, pl.ds(c0, TCOLS)], slab[b], si[b]
        )

    def transpose_chunk(b, ncols):
        @plsc.parallel_loop(0, ncols // 16, unroll=4)
        def _(r0g):
            r0 = r0g * 16
            addr0 = lane33 + r0 * SKEW
            for c in range(EMBEDDING_DIM):
                v = slab[b][c, pl.ds(r0, 16)]
                plsc.store_scatter(skew, [addr0 + c], v)

        @plsc.parallel_loop(0, ncols, unroll=8)
        def _(r):
            for g in range(EMBEDDING_DIM // 16):
                v = skew[pl.ds(r * SKEW + 16 * g, 16)]
                tbuf[b][pl.ds(r * EMBEDDING_DIM + 16 * g, 16)] = v

    def fire_store(i, b):
        off = (wid + NUM_WORKERS * i) * (TCOLS * EMBEDDING_DIM)
        return pltpu.async_copy(
            tbuf[b], wlin_hbm.at[pl.ds(off, TCOLS * EMBEDDING_DIM)], so[b]
        )

    def wait_load(b):
        pltpu.make_async_copy(wt_hbm.at[:, pl.ds(0, TCOLS)], slab[b], si[b]).wait()

    def wait_store(b):
        pltpu.make_async_copy(
            tbuf[b], wlin_hbm.at[pl.ds(0, TCOLS * EMBEDDING_DIM)], so[b]
        ).wait()

    fire_load(0, 0)
    fire_load(1, 1)

    def chunk_body(c, carry):
        def process(b):
            @pl.when(c >= 2)
            def _():
                wait_store(b)

            wait_load(b)
            transpose_chunk(b, TCOLS)

            @pl.when(c + 2 < NPIPE)
            def _():
                fire_load(c + 2, b)

            fire_store(c, b)

        @pl.when(lax.rem(c, 2) == 0)
        def _():
            process(0)

        @pl.when(lax.rem(c, 2) == 1)
        def _():
            process(1)

        return carry

    lax.fori_loop(0, NPIPE, chunk_body, 0)
    for b in range(2):
        wait_store(b)

    @pl.when(wid == 0)
    def _():
        pltpu.sync_copy(
            wt_hbm.at[:, pl.ds(EXTRA_CHUNK * TCOLS, TCOLS)], slab[0]
        )
        transpose_chunk(0, TCOLS)
        pltpu.sync_copy(
            tbuf[0],
            wlin_hbm.at[
                pl.ds(EXTRA_CHUNK * TCOLS * EMBEDDING_DIM, TCOLS * EMBEDDING_DIM)
            ],
        )

    @pl.when(wid == 1)
    def _():
        pltpu.sync_copy(
            tail_hbm, tbuf[0].at[pl.ds(0, TAIL * EMBEDDING_DIM)]
        )
        pltpu.sync_copy(
            tbuf[0].at[pl.ds(0, TAIL * EMBEDDING_DIM)],
            wlin_hbm.at[pl.ds(FULL_COLS * EMBEDDING_DIM, TAIL * EMBEDDING_DIM)],
        )


B_PER_W = BATCH // NUM_WORKERS
GB = 64
GCHUNK = GB * FIELDS
NGC = B_PER_W // GB
STG_MINOR = GB + 1


@functools.partial(
    pl.kernel,
    mesh=_mesh,
    out_type=jax.ShapeDtypeStruct(
        (FIELDS, 4, BATCH // 128, 8, 128), jnp.float32
    ),
    scratch_types=[
        pltpu.VMEM((GCHUNK,), jnp.int32),
        pltpu.VMEM((GCHUNK, EMBEDDING_DIM), jnp.float32),
        pltpu.VMEM((FIELDS, 4, 8, STG_MINOR), jnp.float32),
        pltpu.SemaphoreType.DMA,
        pltpu.SemaphoreType.DMA,
    ],
    compiler_params=pltpu.CompilerParams(
        use_tc_tiling_on_sc=False, needs_layout_passes=False
    ),
)
def _gather_kernel(idx_hbm, table_hbm, out_hbm, idx_v, rows_v, stg_v, sg, sw):
    wid = lax.axis_index("s") * 2 + lax.axis_index("c")
    lane = jax.lax.broadcasted_iota(jnp.int32, (16,), 0)
    ci_vec = lane & 7
    br_vec = (lane >> 3, 2 + (lane >> 3))

    def fire_gather(c):
        j0 = (wid * B_PER_W + c * GB) * FIELDS
        pltpu.sync_copy(idx_hbm.at[pl.ds(j0, GCHUNK)], idx_v)
        return pltpu.async_copy(table_hbm.at[idx_v], rows_v, sg)

    def wait_gather():
        pltpu.make_async_copy(table_hbm.at[idx_v], rows_v, sg).wait()

    fire_gather(0)

    def chunk_body(c, carry):
        wait_gather()

        @plsc.parallel_loop(0, GB, unroll=2)
        def _(bi):
            bi_vec = lane * 0 + bi
            for f in range(FIELDS):
                f_vec = lane * 0 + f
                for g in range(2):
                    v = rows_v[bi * FIELDS + f, pl.ds(16 * g, 16)]
                    plsc.store_scatter(
                        stg_v, [f_vec, br_vec[g], ci_vec, bi_vec], v
                    )

        bc = wid * (B_PER_W // 128) + c // 2
        bi0 = (c % 2) * GB
        copies = []
        for f in range(FIELDS):
            copies.append(
                pltpu.async_copy(
                    stg_v.at[f, :, :, pl.ds(0, GB)],
                    out_hbm.at[f, :, bc, :, pl.ds(bi0, GB)],
                    sw,
                )
            )

        @pl.when(c + 1 < NGC)
        def _():
            fire_gather(c + 1)

        for cp in copies:
            cp.wait()
        return carry

    lax.fori_loop(0, NGC, chunk_body, 0)


def kernel(x, weight):
    tail = weight[FULL_COLS:].reshape(-1)
    wlin = _transpose_kernel(weight.T, tail)
    table = wlin.reshape(NUM_EMB, EMBEDDING_DIM)
    idx = x.reshape(-1)
    o5 = _gather_kernel(idx, table)
    return o5.transpose(2, 4, 0, 1, 3).reshape(BATCH, FIELDS, EMBEDDING_DIM)
